# Initial kernel scaffold; baseline (speedup 1.0000x reference)
import jax
import jax.numpy as jnp
from jax import lax
from jax.experimental import pallas as pl
from jax.experimental.pallas import tpu as pltpu
from jax.experimental.pallas import tpu_sc as plsc

_B = 16384
_XD = 64
_D = 10
_NT = 5
_OUT_D = _XD + _NT * _D

_NC = 2
_NS = 16
_NW = _NC * _NS
_BPW = _B // _NW
_CHUNK = 128
_NCH = _BPW // _CHUNK


def _body(x_hbm, yt_hbm, t0, t1, t2, t3, t4, out_hbm, xv, idxv, rowsv, sem):
    wid = lax.axis_index("s") * _NC + lax.axis_index("c")
    base = wid * _BPW

    pltpu.sync_copy(x_hbm.at[pl.ds(base, _BPW)], xv)
    pltpu.sync_copy(xv, out_hbm.at[pl.ds(base, _BPW), pl.ds(0, _XD)])

    for i, t in enumerate((t0, t1, t2, t3, t4)):
        pltpu.sync_copy(yt_hbm.at[i, pl.ds(base, _BPW)], idxv)
        for j in range(_NCH):
            pltpu.async_copy(
                t.at[idxv.at[pl.ds(j * _CHUNK, _CHUNK)]],
                rowsv.at[pl.ds(j * _CHUNK, _CHUNK)],
                sem,
            ).wait()
        pltpu.sync_copy(
            rowsv,
            out_hbm.at[pl.ds(base, _BPW), pl.ds(_XD + i * _D, _D)],
        )


_sc_call = pl.kernel(
    _body,
    out_type=jax.ShapeDtypeStruct((_B, _OUT_D), jnp.float32),
    mesh=plsc.VectorSubcoreMesh(core_axis_name="c", subcore_axis_name="s"),
    scratch_types=[
        pltpu.VMEM((_BPW, _XD), jnp.float32),
        pltpu.VMEM((_BPW,), jnp.int32),
        pltpu.VMEM((_BPW, _D), jnp.float32),
        pltpu.SemaphoreType.DMA,
    ],
)


def kernel(x, y, emb0, emb1, emb2, emb3, emb4):
    yt = y.astype(jnp.int32).T
    return _sc_call(x, yt, emb0, emb1, emb2, emb3, emb4)


# speedup vs baseline: 1.6682x; 1.6682x over previous
import jax
import jax.numpy as jnp
from jax import lax
from jax.experimental import pallas as pl
from jax.experimental.pallas import tpu as pltpu
from jax.experimental.pallas import tpu_sc as plsc

_B = 16384
_XD = 64
_D = 10
_DP = 16
_NT = 5
_OW = _XD + _NT * _D

_NC = 2
_NS = 16
_NW = _NC * _NS
_R = 128
_NPASS = _B // (_NW * _R)
_NBLK = _B // _R
_OB = _R * _OW
_UNROLL = 4


def _body(x_hbm, yb_hbm, t0, t1, t2, t3, t4, out_hbm, xv, idxv, rows, outv, sem):
    wid = lax.axis_index("s") * _NC + lax.axis_index("c")
    tables = (t0, t1, t2, t3, t4)

    for p in range(_NPASS):
        blk = wid * _NPASS + p
        pltpu.sync_copy(x_hbm.at[pl.ds(blk * (_R * _XD), _R * _XD)], xv)
        pltpu.sync_copy(yb_hbm.at[blk], idxv)
        cps = [
            pltpu.make_async_copy(tables[i].at[idxv.at[i]], rows.at[i], sem)
            for i in range(_NT)
        ]
        for c in cps:
            c.start()
        for c in cps:
            c.wait()

        def assemble(it, _):
            for u in range(_UNROLL):
                r = it * _UNROLL + u
                ob = r * _OW
                xb = r * _XD
                for c in range(_XD // 16):
                    outv[pl.ds(ob + 16 * c, 16)] = xv[pl.ds(xb + 16 * c, 16)]
                for i in range(_NT):
                    outv[pl.ds(ob + _XD + _D * i, 16)] = rows[i, r, :]
            return ()

        lax.fori_loop(0, _R // _UNROLL, assemble, (), unroll=2)

        pltpu.sync_copy(outv.at[pl.ds(0, _OB)], out_hbm.at[pl.ds(blk * _OB, _OB)])


_sc_call = pl.kernel(
    _body,
    out_type=jax.ShapeDtypeStruct((_B * _OW,), jnp.float32),
    mesh=plsc.VectorSubcoreMesh(core_axis_name="c", subcore_axis_name="s"),
    scratch_types=[
        pltpu.VMEM((_R * _XD,), jnp.float32),
        pltpu.VMEM((_NT, _R), jnp.int32),
        pltpu.VMEM((_NT, _R, _DP), jnp.float32),
        pltpu.VMEM((_OB + 16,), jnp.float32),
        pltpu.SemaphoreType.DMA,
    ],
    compiler_params=pltpu.CompilerParams(use_tc_tiling_on_sc=False),
)


def kernel(x, y, emb0, emb1, emb2, emb3, emb4):
    yb = y.astype(jnp.int32).reshape(_NBLK, _R, _NT).transpose(0, 2, 1)
    tables = [
        jnp.pad(t, ((0, 0), (0, _DP - _D)))
        for t in (emb0, emb1, emb2, emb3, emb4)
    ]
    out = _sc_call(x.reshape(-1), yb, *tables)
    return out.reshape(_B, _OW)
